# baseline (speedup 1.0000x reference)
import functools

import jax
import jax.numpy as jnp
from jax import lax
from jax.experimental import pallas as pl
from jax.experimental.pallas import tpu as pltpu
from jax.experimental.pallas import tpu_sc as plsc

N = 10000
E = 320000
D = 128
DE = 16
H = 128
DEPTH = 3

NC = 2
NS = 16
NW = NC * NS
CHUNK = 128
NCH = 80
GC = 16
NG = NCH // GC
EPT = CHUNK * NCH
EP = EPT * NW
ACC = 10112
RPT = ACC // NS
TRASH = N + 8


def _gs_body(table, src_i, dst_i, init, out, src_v, dst_v, rows_v, acc,
             sem0, sem1):
    c = lax.axis_index("c")
    s = lax.axis_index("s")
    wid = c * NS + s
    pltpu.sync_copy(init.at[c, pl.ds(s * RPT, RPT)], acc.at[pl.ds(s * RPT, RPT)])
    plsc.subcore_barrier()

    sems = (sem0, sem1)

    def group(g, carry):
        pltpu.sync_copy(src_i.at[wid, pl.ds(g * GC, GC)], src_v)
        pltpu.sync_copy(dst_i.at[wid, pl.ds(g * GC, GC)], dst_v)
        pltpu.async_copy(table.at[src_v.at[0]], rows_v.at[0], sem0)
        pltpu.async_copy(table.at[src_v.at[1]], rows_v.at[1], sem1)

        def body(j, carry2):
            for b in range(2):
                k = 2 * j + b
                pltpu.make_async_copy(table.at[src_v.at[k]], rows_v.at[b],
                                      sems[b]).wait()
                pltpu.sync_copy(rows_v.at[b], acc.at[dst_v.at[k]], add=True)

                @pl.when(k + 2 < GC)
                def _():
                    pltpu.async_copy(table.at[src_v.at[k + 2]], rows_v.at[b],
                                     sems[b])
            return carry2

        lax.fori_loop(0, GC // 2, body, 0)
        return carry

    lax.fori_loop(0, NG, group, 0)
    plsc.subcore_barrier()
    pltpu.sync_copy(acc.at[pl.ds(s * RPT, RPT)],
                    out.at[c, pl.ds(s * RPT, RPT)])


def _gs8_body(table, gid_i, dst_i, init, out, gid_v, dst_v, rows_v, acc,
              sem0, sem1):
    c = lax.axis_index("c")
    s = lax.axis_index("s")
    wid = c * NS + s
    pltpu.sync_copy(init.at[c, pl.ds(s * RPT, RPT)], acc.at[pl.ds(s * RPT, RPT)])
    plsc.subcore_barrier()

    sems = (sem0, sem1)

    def group(g, carry):
        pltpu.sync_copy(gid_i.at[wid, pl.ds(g * GC, GC)], gid_v)
        pltpu.sync_copy(dst_i.at[wid, pl.ds(g * GC, GC)], dst_v)
        pltpu.async_copy(table.at[gid_v.at[0]], rows_v.at[0], sem0)
        pltpu.async_copy(table.at[gid_v.at[1]], rows_v.at[1], sem1)

        def body(j, carry2):
            for b in range(2):
                k = 2 * j + b
                pltpu.make_async_copy(table.at[gid_v.at[k]], rows_v.at[b],
                                      sems[b]).wait()
                pltpu.sync_copy(rows_v.at[b].reshape(CHUNK, H),
                                acc.at[dst_v.at[k]], add=True)

                @pl.when(k + 2 < GC)
                def _():
                    pltpu.async_copy(table.at[gid_v.at[k + 2]], rows_v.at[b],
                                     sems[b])
            return carry2

        lax.fori_loop(0, GC // 2, body, 0)
        return carry

    lax.fori_loop(0, NG, group, 0)
    plsc.subcore_barrier()
    pltpu.sync_copy(acc.at[pl.ds(s * RPT, RPT)],
                    out.at[c, pl.ds(s * RPT, RPT)])


def _make_gs8():
    mesh = plsc.VectorSubcoreMesh(core_axis_name="c", subcore_axis_name="s")
    return pl.kernel(
        _gs8_body,
        out_type=jax.ShapeDtypeStruct((NC, ACC, H), jnp.float32),
        mesh=mesh,
        scratch_types=[
            pltpu.VMEM((GC, CHUNK // 8), jnp.int32),
            pltpu.VMEM((GC, CHUNK), jnp.int32),
            pltpu.VMEM((2, CHUNK // 8, 8, H), jnp.float32),
            pltpu.VMEM_SHARED((ACC, H), jnp.float32),
            pltpu.SemaphoreType.DMA,
            pltpu.SemaphoreType.DMA,
        ],
    )


def _make_gs(table_rows):
    mesh = plsc.VectorSubcoreMesh(core_axis_name="c", subcore_axis_name="s")
    return pl.kernel(
        _gs_body,
        out_type=jax.ShapeDtypeStruct((NC, ACC, H), jnp.float32),
        mesh=mesh,
        scratch_types=[
            pltpu.VMEM((GC, CHUNK), jnp.int32),
            pltpu.VMEM((GC, CHUNK), jnp.int32),
            pltpu.VMEM((2, CHUNK, H), jnp.float32),
            pltpu.VMEM_SHARED((ACC, H), jnp.float32),
            pltpu.SemaphoreType.DMA,
            pltpu.SemaphoreType.DMA,
        ],
    )


def _mm_relu_body(x_ref, w_ref, o_ref):
    o_ref[...] = jax.nn.relu(
        jnp.dot(x_ref[...], w_ref[...], preferred_element_type=jnp.float32))


def _mm_relu(x, w, blk, out_rows=None):
    m, k = x.shape
    _, n = w.shape
    return pl.pallas_call(
        _mm_relu_body,
        grid=(m // blk,),
        in_specs=[
            pl.BlockSpec((blk, k), lambda i: (i, 0)),
            pl.BlockSpec((k, n), lambda i: (0, 0)),
        ],
        out_specs=pl.BlockSpec((blk, n), lambda i: (i, 0)),
        out_shape=jax.ShapeDtypeStruct((out_rows or m, n), jnp.float32),
    )(x, w)


def _e_body(blk, ea_ref, w_ref, o_ref, xbuf, sem):
    i = pl.program_id(0)
    pltpu.async_copy(ea_ref.at[pl.ds(i * blk, blk), :], xbuf, sem).wait()
    o_ref[...] = jax.nn.relu(
        jnp.dot(xbuf[...], w_ref[...], preferred_element_type=jnp.float32))


def _e_embed(ea, w, blk):
    m = ea.shape[0]
    return pl.pallas_call(
        functools.partial(_e_body, blk),
        grid=(m // blk,),
        in_specs=[
            pl.BlockSpec(memory_space=pltpu.MemorySpace.HBM),
            pl.BlockSpec((DE, H), lambda i: (0, 0)),
        ],
        out_specs=pl.BlockSpec((blk, H), lambda i: (i, 0)),
        out_shape=jax.ShapeDtypeStruct((EP, H), jnp.float32),
        scratch_shapes=[
            pltpu.VMEM((blk, DE), jnp.float32),
            pltpu.SemaphoreType.DMA,
        ],
    )(ea, w)


def _upd_body(h_ref, p_ref, w_ref, h_out, m_out):
    m = p_ref[0] + p_ref[1]
    m_out[...] = m
    h_out[...] = jax.nn.relu(
        jnp.dot(h_ref[...], w_ref[...], preferred_element_type=jnp.float32) + m)


def _update(h, parts, w, blk=2000):
    return pl.pallas_call(
        _upd_body,
        grid=(N // blk,),
        in_specs=[
            pl.BlockSpec((blk, H), lambda i: (i, 0)),
            pl.BlockSpec((NC, blk, H), lambda i: (0, i, 0)),
            pl.BlockSpec((H, H), lambda i: (0, 0)),
        ],
        out_specs=[
            pl.BlockSpec((blk, H), lambda i: (i, 0)),
            pl.BlockSpec((blk, H), lambda i: (i, 0)),
        ],
        out_shape=[
            jax.ShapeDtypeStruct((N, H), jnp.float32),
            jax.ShapeDtypeStruct((N, H), jnp.float32),
        ],
    )(h, parts, w)


def _head_body(h_ref, w1_ref, b1_ref, w2_ref, b2_ref, o_ref):
    a = jax.nn.relu(
        jnp.dot(h_ref[...], w1_ref[...], preferred_element_type=jnp.float32)
        + b1_ref[...])
    o_ref[...] = (jnp.dot(a, w2_ref[...], preferred_element_type=jnp.float32)
                  + b2_ref[...])


def _head(h, w1, b1, w2, b2, blk=2000):
    ffn_h = w1.shape[1]
    out = w2.shape[1]
    return pl.pallas_call(
        _head_body,
        grid=(N // blk,),
        in_specs=[
            pl.BlockSpec((blk, H), lambda i: (i, 0)),
            pl.BlockSpec((H, ffn_h), lambda i: (0, 0)),
            pl.BlockSpec((1, ffn_h), lambda i: (0, 0)),
            pl.BlockSpec((ffn_h, out), lambda i: (0, 0)),
            pl.BlockSpec((1, out), lambda i: (0, 0)),
        ],
        out_specs=pl.BlockSpec((blk, out), lambda i: (i, 0)),
        out_shape=jax.ShapeDtypeStruct((N, out), jnp.float32),
    )(h, w1, b1.reshape(1, ffn_h), w2, b2.reshape(1, out))


def kernel(x, edge_attr, W_in, W_h, W_e, ffn_W1, ffn_b1, ffn_W2, ffn_b2,
           edge_index):
    src = edge_index[0]
    dst = edge_index[1]
    pad = EP - E
    src_p = jnp.concatenate(
        [src, jnp.arange(pad, dtype=jnp.int32) % N]).reshape(NW, NCH, CHUNK)
    dst_p = jnp.concatenate([dst, jnp.full((pad,), TRASH, jnp.int32)]).reshape(
        NW, NCH, CHUNK)
    iota_p = jnp.concatenate(
        [jnp.arange(E, dtype=jnp.int32),
         jnp.arange(pad, dtype=jnp.int32)]).reshape(NW, NCH, CHUNK)

    h = _mm_relu(x, W_in, blk=2000)
    e_p = _e_embed(edge_attr, W_e, blk=3200)

    zero_init = jnp.zeros((NC, ACC, H), jnp.float32)
    gs_e = _make_gs(EP)
    gs_h = _make_gs(N)
    parts_e = gs_e(e_p, iota_p, dst_p, zero_init)

    m = None
    for _ in range(DEPTH):
        parts = gs_h(h, src_p, dst_p, parts_e)
        h, m = _update(h, parts, W_h)

    r = _head(h, ffn_W1, ffn_b1, ffn_W2, ffn_b2)
    return (r, m, h)


# speedup vs baseline: 1.2666x; 1.2666x over previous
import functools

import jax
import jax.numpy as jnp
from jax import lax
from jax.experimental import pallas as pl
from jax.experimental.pallas import tpu as pltpu
from jax.experimental.pallas import tpu_sc as plsc

N = 10000
E = 320000
D = 128
DE = 16
H = 128
DEPTH = 3

NC = 2
NS = 16
NW = NC * NS
CHUNK = 128
NCH = 80
GC = 16
NG = NCH // GC
EPT = CHUNK * NCH
EP = EPT * NW
ACC = 10112
RPT = ACC // NS
TRASH = N + 8


def _gs_body(table, src_i, dst_i, init, out, src_v, dst_v, rows_v, acc,
             sem0, sem1):
    c = lax.axis_index("c")
    s = lax.axis_index("s")
    wid = c * NS + s
    pltpu.sync_copy(init.at[c, pl.ds(s * RPT, RPT)], acc.at[pl.ds(s * RPT, RPT)])
    plsc.subcore_barrier()

    sems = (sem0, sem1)

    def group(g, carry):
        pltpu.sync_copy(src_i.at[wid, pl.ds(g * GC, GC)], src_v)
        pltpu.sync_copy(dst_i.at[wid, pl.ds(g * GC, GC)], dst_v)
        pltpu.async_copy(table.at[src_v.at[0]], rows_v.at[0], sem0)
        pltpu.async_copy(table.at[src_v.at[1]], rows_v.at[1], sem1)

        def body(j, carry2):
            for b in range(2):
                k = 2 * j + b
                pltpu.make_async_copy(table.at[src_v.at[k]], rows_v.at[b],
                                      sems[b]).wait()
                pltpu.sync_copy(rows_v.at[b], acc.at[dst_v.at[k]], add=True)

                @pl.when(k + 2 < GC)
                def _():
                    pltpu.async_copy(table.at[src_v.at[k + 2]], rows_v.at[b],
                                     sems[b])
            return carry2

        lax.fori_loop(0, GC // 2, body, 0)
        return carry

    lax.fori_loop(0, NG, group, 0)
    plsc.subcore_barrier()
    pltpu.sync_copy(acc.at[pl.ds(s * RPT, RPT)],
                    out.at[c, pl.ds(s * RPT, RPT)])


def _gs8_body(table, gid_i, dst_i, init, out, gid_v, dst_v, rows_v, acc,
              sem0, sem1):
    c = lax.axis_index("c")
    s = lax.axis_index("s")
    wid = c * NS + s
    pltpu.sync_copy(init.at[c, pl.ds(s * RPT, RPT)], acc.at[pl.ds(s * RPT, RPT)])
    plsc.subcore_barrier()

    sems = (sem0, sem1)

    def group(g, carry):
        pltpu.sync_copy(gid_i.at[wid, pl.ds(g * GC, GC)], gid_v)
        pltpu.sync_copy(dst_i.at[wid, pl.ds(g * GC, GC)], dst_v)
        pltpu.async_copy(table.at[gid_v.at[0]], rows_v.at[0], sem0)
        pltpu.async_copy(table.at[gid_v.at[1]], rows_v.at[1], sem1)

        def body(j, carry2):
            for b in range(2):
                k = 2 * j + b
                pltpu.make_async_copy(table.at[gid_v.at[k]], rows_v.at[b],
                                      sems[b]).wait()
                pltpu.sync_copy(rows_v.at[b].reshape(CHUNK, H),
                                acc.at[dst_v.at[k]], add=True)

                @pl.when(k + 2 < GC)
                def _():
                    pltpu.async_copy(table.at[gid_v.at[k + 2]], rows_v.at[b],
                                     sems[b])
            return carry2

        lax.fori_loop(0, GC // 2, body, 0)
        return carry

    lax.fori_loop(0, NG, group, 0)
    plsc.subcore_barrier()
    pltpu.sync_copy(acc.at[pl.ds(s * RPT, RPT)],
                    out.at[c, pl.ds(s * RPT, RPT)])


def _make_gs8():
    mesh = plsc.VectorSubcoreMesh(core_axis_name="c", subcore_axis_name="s")
    return pl.kernel(
        _gs8_body,
        out_type=jax.ShapeDtypeStruct((NC, ACC, H), jnp.float32),
        mesh=mesh,
        scratch_types=[
            pltpu.VMEM((GC, CHUNK // 8), jnp.int32),
            pltpu.VMEM((GC, CHUNK), jnp.int32),
            pltpu.VMEM((2, CHUNK // 8, 8, H), jnp.float32),
            pltpu.VMEM_SHARED((ACC, H), jnp.float32),
            pltpu.SemaphoreType.DMA,
            pltpu.SemaphoreType.DMA,
        ],
    )


def _make_gs(table_rows):
    mesh = plsc.VectorSubcoreMesh(core_axis_name="c", subcore_axis_name="s")
    return pl.kernel(
        _gs_body,
        out_type=jax.ShapeDtypeStruct((NC, ACC, H), jnp.float32),
        mesh=mesh,
        scratch_types=[
            pltpu.VMEM((GC, CHUNK), jnp.int32),
            pltpu.VMEM((GC, CHUNK), jnp.int32),
            pltpu.VMEM((2, CHUNK, H), jnp.float32),
            pltpu.VMEM_SHARED((ACC, H), jnp.float32),
            pltpu.SemaphoreType.DMA,
            pltpu.SemaphoreType.DMA,
        ],
    )


def _mm_relu_body(x_ref, w_ref, o_ref):
    o_ref[...] = jax.nn.relu(
        jnp.dot(x_ref[...], w_ref[...], preferred_element_type=jnp.float32))


def _mm_relu(x, w, blk, out_rows=None):
    m, k = x.shape
    _, n = w.shape
    return pl.pallas_call(
        _mm_relu_body,
        grid=(m // blk,),
        in_specs=[
            pl.BlockSpec((blk, k), lambda i: (i, 0)),
            pl.BlockSpec((k, n), lambda i: (0, 0)),
        ],
        out_specs=pl.BlockSpec((blk, n), lambda i: (i, 0)),
        out_shape=jax.ShapeDtypeStruct((out_rows or m, n), jnp.float32),
    )(x, w)


def _upd4_body(h_ref, ph_ref, pe_ref, w_ref, h_out, m_out):
    m = ph_ref[0] + ph_ref[1] + pe_ref[0] + pe_ref[1]
    m_out[...] = m
    h_out[...] = jax.nn.relu(
        jnp.dot(h_ref[...], w_ref[...], preferred_element_type=jnp.float32) + m)


def _update4(h, parts_h, parts_e, w, blk=2000):
    return pl.pallas_call(
        _upd4_body,
        grid=(N // blk,),
        in_specs=[
            pl.BlockSpec((blk, H), lambda i: (i, 0)),
            pl.BlockSpec((NC, blk, H), lambda i: (0, i, 0)),
            pl.BlockSpec((NC, blk, H), lambda i: (0, i, 0)),
            pl.BlockSpec((H, H), lambda i: (0, 0)),
        ],
        out_specs=[
            pl.BlockSpec((blk, H), lambda i: (i, 0)),
            pl.BlockSpec((blk, H), lambda i: (i, 0)),
        ],
        out_shape=[
            jax.ShapeDtypeStruct((N, H), jnp.float32),
            jax.ShapeDtypeStruct((N, H), jnp.float32),
        ],
    )(h, parts_h, parts_e, w)


def _upd_body(h_ref, p_ref, w_ref, h_out, m_out):
    m = p_ref[0] + p_ref[1]
    m_out[...] = m
    h_out[...] = jax.nn.relu(
        jnp.dot(h_ref[...], w_ref[...], preferred_element_type=jnp.float32) + m)


def _update(h, parts, w, blk=2000):
    return pl.pallas_call(
        _upd_body,
        grid=(N // blk,),
        in_specs=[
            pl.BlockSpec((blk, H), lambda i: (i, 0)),
            pl.BlockSpec((NC, blk, H), lambda i: (0, i, 0)),
            pl.BlockSpec((H, H), lambda i: (0, 0)),
        ],
        out_specs=[
            pl.BlockSpec((blk, H), lambda i: (i, 0)),
            pl.BlockSpec((blk, H), lambda i: (i, 0)),
        ],
        out_shape=[
            jax.ShapeDtypeStruct((N, H), jnp.float32),
            jax.ShapeDtypeStruct((N, H), jnp.float32),
        ],
    )(h, parts, w)


def _head_body(h_ref, w1_ref, b1_ref, w2_ref, b2_ref, o_ref):
    a = jax.nn.relu(
        jnp.dot(h_ref[...], w1_ref[...], preferred_element_type=jnp.float32)
        + b1_ref[...])
    o_ref[...] = (jnp.dot(a, w2_ref[...], preferred_element_type=jnp.float32)
                  + b2_ref[...])


def _head(h, w1, b1, w2, b2, blk=2000):
    ffn_h = w1.shape[1]
    out = w2.shape[1]
    return pl.pallas_call(
        _head_body,
        grid=(N // blk,),
        in_specs=[
            pl.BlockSpec((blk, H), lambda i: (i, 0)),
            pl.BlockSpec((H, ffn_h), lambda i: (0, 0)),
            pl.BlockSpec((1, ffn_h), lambda i: (0, 0)),
            pl.BlockSpec((ffn_h, out), lambda i: (0, 0)),
            pl.BlockSpec((1, out), lambda i: (0, 0)),
        ],
        out_specs=pl.BlockSpec((blk, out), lambda i: (i, 0)),
        out_shape=jax.ShapeDtypeStruct((N, out), jnp.float32),
    )(h, w1, b1.reshape(1, ffn_h), w2, b2.reshape(1, out))


def kernel(x, edge_attr, W_in, W_h, W_e, ffn_W1, ffn_b1, ffn_W2, ffn_b2,
           edge_index):
    src = edge_index[0]
    dst = edge_index[1]
    pad = EP - E
    src_p = jnp.concatenate(
        [src, jnp.arange(pad, dtype=jnp.int32) % N]).reshape(NW, NCH, CHUNK)
    dst_p = jnp.concatenate([dst, jnp.full((pad,), TRASH, jnp.int32)]).reshape(
        NW, NCH, CHUNK)
    iota_p = jnp.concatenate(
        [jnp.arange(E, dtype=jnp.int32),
         jnp.arange(pad, dtype=jnp.int32)]).reshape(NW, NCH, CHUNK)

    h = _mm_relu(x, W_in, blk=2000)
    e_p = _mm_relu(edge_attr, W_e, blk=3200, out_rows=EP)

    zero_init = jnp.zeros((NC, ACC, H), jnp.float32)
    gs_e = _make_gs(EP)
    gs_h = _make_gs(N)
    parts_e = gs_e(e_p, iota_p, dst_p, zero_init)

    m = None
    for _ in range(DEPTH):
        parts_h = gs_h(h, src_p, dst_p, zero_init)
        h, m = _update4(h, parts_h, parts_e, W_h)

    r = _head(h, ffn_W1, ffn_b1, ffn_W2, ffn_b2)
    return (r, m, h)


# speedup vs baseline: 1.3513x; 1.0669x over previous
import functools

import jax
import jax.numpy as jnp
from jax import lax
from jax.experimental import pallas as pl
from jax.experimental.pallas import tpu as pltpu
from jax.experimental.pallas import tpu_sc as plsc

N = 10000
E = 320000
D = 128
DE = 16
H = 128
DEPTH = 3

NC = 2
NS = 16
NW = NC * NS
CHUNK = 128
NCH = 80
GC = 16
NG = NCH // GC
EPT = CHUNK * NCH
EP = EPT * NW
ACC = 10112
RPT = ACC // NS
TRASH = N + 8


def _gs_body(table, src_i, dst_i, init, out, src_v, dst_v, rows_v, acc,
             sem0, sem1):
    c = lax.axis_index("c")
    s = lax.axis_index("s")
    wid = c * NS + s
    pltpu.sync_copy(init.at[c, pl.ds(s * RPT, RPT)], acc.at[pl.ds(s * RPT, RPT)])
    plsc.subcore_barrier()

    sems = (sem0, sem1)

    def group(g, carry):
        pltpu.sync_copy(src_i.at[wid, pl.ds(g * GC, GC)], src_v)
        pltpu.sync_copy(dst_i.at[wid, pl.ds(g * GC, GC)], dst_v)
        pltpu.async_copy(table.at[src_v.at[0]], rows_v.at[0], sem0)
        pltpu.async_copy(table.at[src_v.at[1]], rows_v.at[1], sem1)

        def body(j, carry2):
            for b in range(2):
                k = 2 * j + b
                pltpu.make_async_copy(table.at[src_v.at[k]], rows_v.at[b],
                                      sems[b]).wait()
                pltpu.sync_copy(rows_v.at[b], acc.at[dst_v.at[k]], add=True)

                @pl.when(k + 2 < GC)
                def _():
                    pltpu.async_copy(table.at[src_v.at[k + 2]], rows_v.at[b],
                                     sems[b])
            return carry2

        lax.fori_loop(0, GC // 2, body, 0)
        return carry

    lax.fori_loop(0, NG, group, 0)
    plsc.subcore_barrier()
    pltpu.sync_copy(acc.at[pl.ds(s * RPT, RPT)],
                    out.at[c, pl.ds(s * RPT, RPT)])


def _gs8_body(table, gid_i, dst_i, init, out, gid_v, dst_v, rows_v, acc,
              sem0, sem1):
    c = lax.axis_index("c")
    s = lax.axis_index("s")
    wid = c * NS + s
    pltpu.sync_copy(init.at[c, pl.ds(s * RPT, RPT)], acc.at[pl.ds(s * RPT, RPT)])
    plsc.subcore_barrier()

    sems = (sem0, sem1)

    def group(g, carry):
        pltpu.sync_copy(gid_i.at[wid, pl.ds(g * GC, GC)], gid_v)
        pltpu.sync_copy(dst_i.at[wid, pl.ds(g * GC, GC)], dst_v)
        pltpu.async_copy(table.at[gid_v.at[0]], rows_v.at[0], sem0)
        pltpu.async_copy(table.at[gid_v.at[1]], rows_v.at[1], sem1)

        def body(j, carry2):
            for b in range(2):
                k = 2 * j + b
                pltpu.make_async_copy(table.at[gid_v.at[k]], rows_v.at[b],
                                      sems[b]).wait()
                pltpu.sync_copy(rows_v.at[b].reshape(CHUNK, H),
                                acc.at[dst_v.at[k]], add=True)

                @pl.when(k + 2 < GC)
                def _():
                    pltpu.async_copy(table.at[gid_v.at[k + 2]], rows_v.at[b],
                                     sems[b])
            return carry2

        lax.fori_loop(0, GC // 2, body, 0)
        return carry

    lax.fori_loop(0, NG, group, 0)
    plsc.subcore_barrier()
    pltpu.sync_copy(acc.at[pl.ds(s * RPT, RPT)],
                    out.at[c, pl.ds(s * RPT, RPT)])


def _make_gs8():
    mesh = plsc.VectorSubcoreMesh(core_axis_name="c", subcore_axis_name="s")
    return pl.kernel(
        _gs8_body,
        out_type=jax.ShapeDtypeStruct((NC, ACC, H), jnp.float32),
        mesh=mesh,
        scratch_types=[
            pltpu.VMEM((GC, CHUNK // 8), jnp.int32),
            pltpu.VMEM((GC, CHUNK), jnp.int32),
            pltpu.VMEM((2, CHUNK // 8, 8, H), jnp.float32),
            pltpu.VMEM_SHARED((ACC, H), jnp.float32),
            pltpu.SemaphoreType.DMA,
            pltpu.SemaphoreType.DMA,
        ],
    )


def _make_gs(table_rows):
    mesh = plsc.VectorSubcoreMesh(core_axis_name="c", subcore_axis_name="s")
    return pl.kernel(
        _gs_body,
        out_type=jax.ShapeDtypeStruct((NC, ACC, H), jnp.float32),
        mesh=mesh,
        scratch_types=[
            pltpu.VMEM((GC, CHUNK), jnp.int32),
            pltpu.VMEM((GC, CHUNK), jnp.int32),
            pltpu.VMEM((2, CHUNK, H), jnp.float32),
            pltpu.VMEM_SHARED((ACC, H), jnp.float32),
            pltpu.SemaphoreType.DMA,
            pltpu.SemaphoreType.DMA,
        ],
    )


def _mm_relu_body(x_ref, w_ref, o_ref):
    o_ref[...] = jax.nn.relu(
        jnp.dot(x_ref[...], w_ref[...], preferred_element_type=jnp.float32))


def _mm_relu(x, w, blk, out_rows=None):
    m, k = x.shape
    _, n = w.shape
    return pl.pallas_call(
        _mm_relu_body,
        grid=(m // blk,),
        in_specs=[
            pl.BlockSpec((blk, k), lambda i: (i, 0)),
            pl.BlockSpec((k, n), lambda i: (0, 0)),
        ],
        out_specs=pl.BlockSpec((blk, n), lambda i: (i, 0)),
        out_shape=jax.ShapeDtypeStruct((out_rows or m, n), jnp.float32),
    )(x, w)


def _upd4_body(h_ref, ph_ref, pe_ref, w_ref, h_out, m_out):
    m = ph_ref[0] + ph_ref[1] + pe_ref[0] + pe_ref[1]
    m_out[...] = m
    h_out[...] = jax.nn.relu(
        jnp.dot(h_ref[...], w_ref[...], preferred_element_type=jnp.float32) + m)


def _update4(h, parts_h, parts_e, w, blk=2000):
    return pl.pallas_call(
        _upd4_body,
        grid=(N // blk,),
        in_specs=[
            pl.BlockSpec((blk, H), lambda i: (i, 0)),
            pl.BlockSpec((NC, blk, H), lambda i: (0, i, 0)),
            pl.BlockSpec((NC, blk, H), lambda i: (0, i, 0)),
            pl.BlockSpec((H, H), lambda i: (0, 0)),
        ],
        out_specs=[
            pl.BlockSpec((blk, H), lambda i: (i, 0)),
            pl.BlockSpec((blk, H), lambda i: (i, 0)),
        ],
        out_shape=[
            jax.ShapeDtypeStruct((N, H), jnp.float32),
            jax.ShapeDtypeStruct((N, H), jnp.float32),
        ],
    )(h, parts_h, parts_e, w)


def _upd_body(h_ref, p_ref, w_ref, h_out, m_out):
    m = p_ref[0] + p_ref[1]
    m_out[...] = m
    h_out[...] = jax.nn.relu(
        jnp.dot(h_ref[...], w_ref[...], preferred_element_type=jnp.float32) + m)


def _update(h, parts, w, blk=2000):
    return pl.pallas_call(
        _upd_body,
        grid=(N // blk,),
        in_specs=[
            pl.BlockSpec((blk, H), lambda i: (i, 0)),
            pl.BlockSpec((NC, blk, H), lambda i: (0, i, 0)),
            pl.BlockSpec((H, H), lambda i: (0, 0)),
        ],
        out_specs=[
            pl.BlockSpec((blk, H), lambda i: (i, 0)),
            pl.BlockSpec((blk, H), lambda i: (i, 0)),
        ],
        out_shape=[
            jax.ShapeDtypeStruct((N, H), jnp.float32),
            jax.ShapeDtypeStruct((N, H), jnp.float32),
        ],
    )(h, parts, w)


def _head_body(h_ref, w1_ref, b1_ref, w2_ref, b2_ref, o_ref):
    a = jax.nn.relu(
        jnp.dot(h_ref[...], w1_ref[...], preferred_element_type=jnp.float32)
        + b1_ref[...])
    o_ref[...] = (jnp.dot(a, w2_ref[...], preferred_element_type=jnp.float32)
                  + b2_ref[...])


def _head(h, w1, b1, w2, b2, blk=2000):
    ffn_h = w1.shape[1]
    out = w2.shape[1]
    return pl.pallas_call(
        _head_body,
        grid=(N // blk,),
        in_specs=[
            pl.BlockSpec((blk, H), lambda i: (i, 0)),
            pl.BlockSpec((H, ffn_h), lambda i: (0, 0)),
            pl.BlockSpec((1, ffn_h), lambda i: (0, 0)),
            pl.BlockSpec((ffn_h, out), lambda i: (0, 0)),
            pl.BlockSpec((1, out), lambda i: (0, 0)),
        ],
        out_specs=pl.BlockSpec((blk, out), lambda i: (i, 0)),
        out_shape=jax.ShapeDtypeStruct((N, out), jnp.float32),
    )(h, w1, b1.reshape(1, ffn_h), w2, b2.reshape(1, out))


def kernel(x, edge_attr, W_in, W_h, W_e, ffn_W1, ffn_b1, ffn_W2, ffn_b2,
           edge_index):
    src = edge_index[0]
    dst = edge_index[1]
    pad = EP - E
    src_p = jnp.concatenate(
        [src, jnp.arange(pad, dtype=jnp.int32) % N]).reshape(NW, NCH, CHUNK)
    dst_p = jnp.concatenate([dst, jnp.full((pad,), TRASH, jnp.int32)]).reshape(
        NW, NCH, CHUNK)
    iota_p = jnp.concatenate(
        [jnp.arange(E, dtype=jnp.int32),
         jnp.arange(pad, dtype=jnp.int32)]).reshape(NW, NCH, CHUNK)

    h = _mm_relu(x, W_in, blk=2000)
    e_p = _mm_relu(edge_attr.astype(jnp.bfloat16),
                   W_e.astype(jnp.bfloat16), blk=3200, out_rows=EP)

    zero_init = jnp.zeros((NC, ACC, H), jnp.float32)
    gs_e = _make_gs(EP)
    gs_h = _make_gs(N)
    parts_e = gs_e(e_p, iota_p, dst_p, zero_init)

    m = None
    for _ in range(DEPTH):
        parts_h = gs_h(h, src_p, dst_p, zero_init)
        h, m = _update4(h, parts_h, parts_e, W_h)

    r = _head(h, ffn_W1, ffn_b1, ffn_W2, ffn_b2)
    return (r, m, h)
